# Initial kernel scaffold; baseline (speedup 1.0000x reference)
import jax
import jax.numpy as jnp
from jax.experimental import pallas as pl


def kernel(src_ids, dst_ids, edge_feat, delta_t, memory, msg_W1, msg_b1, msg_W2, msg_b2, gru_Wih, gru_Whh, gru_bih, gru_bhh, emb_W1, emb_b1, emb_W2, emb_b2, cls_W1, cls_b1, cls_W2, cls_b2):
    raise NotImplementedError("write your pallas kernel here")


# speedup vs baseline: 1.3508x; 1.3508x over previous
import functools

import jax
import jax.numpy as jnp
from jax.experimental import pallas as pl
from jax.experimental.pallas import tpu as pltpu

B = 16384
MD = 128
ED = 16
EMB = 128

BLK = 2048


def _dense_fwd_body(src_ref, dst_ref, ef_ref, dt_ref,
                    w1s_ref, w1d_ref, w1e_ref, w1t_ref, b1_ref,
                    w2_ref, b2_ref,
                    wih_ref, bih_ref, whh_ref, bhh_ref,
                    e1d_ref, e1e_ref, eb1_ref, e2_ref, eb2_ref,
                    c1_ref, cb1_ref, c2_ref, cb2_ref,
                    out_ref, newsrc_ref, gi_ref):
    src = src_ref[...]
    dst = dst_ref[...]
    ef = ef_ref[...]
    dt = dt_ref[...]

    h1 = (jnp.dot(src, w1s_ref[...], preferred_element_type=jnp.float32)
          + jnp.dot(dst, w1d_ref[...], preferred_element_type=jnp.float32)
          + jnp.dot(ef, w1e_ref[...], preferred_element_type=jnp.float32)
          + dt * w1t_ref[...]
          + b1_ref[...])
    h1 = jnp.maximum(h1, 0.0)
    msg = jnp.dot(h1, w2_ref[...], preferred_element_type=jnp.float32) + b2_ref[...]

    gi = jnp.dot(msg, wih_ref[...], preferred_element_type=jnp.float32) + bih_ref[...]
    gi_ref[...] = gi

    gh = jnp.dot(src, whh_ref[...], preferred_element_type=jnp.float32) + bhh_ref[...]
    i_r, i_z, i_n = gi[:, :MD], gi[:, MD:2 * MD], gi[:, 2 * MD:]
    h_r, h_z, h_n = gh[:, :MD], gh[:, MD:2 * MD], gh[:, 2 * MD:]
    r = jax.nn.sigmoid(i_r + h_r)
    z = jax.nn.sigmoid(i_z + h_z)
    n = jnp.tanh(i_n + r * h_n)
    newsrc_ref[...] = (1.0 - z) * n + z * src

    eh = (jnp.dot(dst, e1d_ref[...], preferred_element_type=jnp.float32)
          + jnp.dot(ef, e1e_ref[...], preferred_element_type=jnp.float32)
          + eb1_ref[...])
    eh = jnp.maximum(eh, 0.0)
    embed = jnp.dot(eh, e2_ref[...], preferred_element_type=jnp.float32) + eb2_ref[...]

    ch = jnp.maximum(jnp.dot(embed, c1_ref[...], preferred_element_type=jnp.float32)
                     + cb1_ref[...], 0.0)
    logit = jnp.dot(ch, c2_ref[...], preferred_element_type=jnp.float32) + cb2_ref[...]
    out_ref[...] = jax.nn.sigmoid(logit)


def _gru_dst_body(gi_ref, curd_ref, whh_ref, bhh_ref, newdst_ref):
    gi = gi_ref[...]
    h = curd_ref[...]
    gh = jnp.dot(h, whh_ref[...], preferred_element_type=jnp.float32) + bhh_ref[...]
    i_r, i_z, i_n = gi[:, :MD], gi[:, MD:2 * MD], gi[:, 2 * MD:]
    h_r, h_z, h_n = gh[:, :MD], gh[:, MD:2 * MD], gh[:, 2 * MD:]
    r = jax.nn.sigmoid(i_r + h_r)
    z = jax.nn.sigmoid(i_z + h_z)
    n = jnp.tanh(i_n + r * h_n)
    newdst_ref[...] = (1.0 - z) * n + z * h


def _row_spec(d):
    return pl.BlockSpec((BLK, d), lambda i: (i, 0))


def _full_spec(shape):
    return pl.BlockSpec(shape, lambda i: tuple(0 for _ in shape))


@functools.partial(jax.jit, static_argnums=())
def _dense_fwd(src_mem, dst_mem, ef, dt, weights):
    (w1s, w1d, w1e, w1t, b1, w2, b2, wih, bih, whh, bhh,
     e1d, e1e, eb1, e2, eb2, c1, cb1, c2, cb2) = weights
    grid = (B // BLK,)
    out, newsrc, gi = pl.pallas_call(
        _dense_fwd_body,
        grid=grid,
        in_specs=[
            _row_spec(MD), _row_spec(MD), _row_spec(ED), _row_spec(1),
            _full_spec(w1s.shape), _full_spec(w1d.shape), _full_spec(w1e.shape),
            _full_spec(w1t.shape), _full_spec(b1.shape),
            _full_spec(w2.shape), _full_spec(b2.shape),
            _full_spec(wih.shape), _full_spec(bih.shape),
            _full_spec(whh.shape), _full_spec(bhh.shape),
            _full_spec(e1d.shape), _full_spec(e1e.shape), _full_spec(eb1.shape),
            _full_spec(e2.shape), _full_spec(eb2.shape),
            _full_spec(c1.shape), _full_spec(cb1.shape),
            _full_spec(c2.shape), _full_spec(cb2.shape),
        ],
        out_specs=[_row_spec(1), _row_spec(MD), _row_spec(3 * MD)],
        out_shape=[
            jax.ShapeDtypeStruct((B, 1), jnp.float32),
            jax.ShapeDtypeStruct((B, MD), jnp.float32),
            jax.ShapeDtypeStruct((B, 3 * MD), jnp.float32),
        ],
    )(src_mem, dst_mem, ef, dt,
      w1s, w1d, w1e, w1t, b1, w2, b2, wih, bih, whh, bhh,
      e1d, e1e, eb1, e2, eb2, c1, cb1, c2, cb2)
    return out, newsrc, gi


def _gru_dst(gi, cur_d, whh_t, bhh):
    grid = (B // BLK,)
    return pl.pallas_call(
        _gru_dst_body,
        grid=grid,
        in_specs=[
            _row_spec(3 * MD), _row_spec(MD),
            _full_spec(whh_t.shape), _full_spec(bhh.shape),
        ],
        out_specs=_row_spec(MD),
        out_shape=jax.ShapeDtypeStruct((B, MD), jnp.float32),
    )(gi, cur_d, whh_t, bhh)


def kernel(src_ids, dst_ids, edge_feat, delta_t, memory,
           msg_W1, msg_b1, msg_W2, msg_b2,
           gru_Wih, gru_Whh, gru_bih, gru_bhh,
           emb_W1, emb_b1, emb_W2, emb_b2,
           cls_W1, cls_b1, cls_W2, cls_b2):
    w1 = msg_W1.T
    w1s, w1d = w1[:MD], w1[MD:2 * MD]
    w1e, w1t = w1[2 * MD:2 * MD + ED], w1[2 * MD + ED:]
    weights = (w1s, w1d, w1e, w1t, msg_b1[None, :],
               msg_W2.T, msg_b2[None, :],
               gru_Wih.T, gru_bih[None, :], gru_Whh.T, gru_bhh[None, :],
               emb_W1.T[:MD], emb_W1.T[MD:], emb_b1[None, :],
               emb_W2.T, emb_b2[None, :],
               cls_W1.T, cls_b1[None, :], cls_W2.T, cls_b2[None, :])

    src_mem = jnp.take(memory, src_ids, axis=0)
    dst_mem = jnp.take(memory, dst_ids, axis=0)

    out, new_src, gi = _dense_fwd(src_mem, dst_mem, edge_feat, delta_t, weights)

    mem2 = memory.at[src_ids].set(new_src)
    cur_d = jnp.take(mem2, dst_ids, axis=0)
    new_dst = _gru_dst(gi, cur_d, gru_Whh.T, gru_bhh[None, :])
    mem3 = mem2.at[dst_ids].set(new_dst)
    return out[:, 0], mem3


# speedup vs baseline: 1.4083x; 1.0426x over previous
import functools

import jax
import jax.numpy as jnp
from jax import lax
from jax.experimental import pallas as pl
from jax.experimental.pallas import tpu as pltpu
from jax.experimental.pallas import tpu_sc as plsc

B = 16384
MD = 128
ED = 16
EMB = 128

BLK = 2048

NC = 2
NS = 16
NW = NC * NS
RPW = B // NW

_SC_MESH = plsc.VectorSubcoreMesh(core_axis_name="c", subcore_axis_name="s")


def _sc_wid():
    return lax.axis_index("s") * NC + lax.axis_index("c")


@functools.partial(
    pl.kernel,
    out_type=[
        jax.ShapeDtypeStruct((B, MD), jnp.float32),
        jax.ShapeDtypeStruct((B, MD), jnp.float32),
    ],
    mesh=_SC_MESH,
    scratch_types=[
        pltpu.VMEM((RPW,), jnp.int32),
        pltpu.VMEM((RPW, MD), jnp.float32),
        pltpu.SemaphoreType.DMA,
    ],
)
def _sc_gather2(mem_hbm, src_hbm, dst_hbm, srcmem_out, dstmem_out,
                idx_v, rows_v, sem):
    wid = _sc_wid()
    base = wid * RPW
    pltpu.sync_copy(src_hbm.at[pl.ds(base, RPW)], idx_v)
    pltpu.async_copy(mem_hbm.at[idx_v], rows_v, sem).wait()
    pltpu.sync_copy(rows_v, srcmem_out.at[pl.ds(base, RPW)])
    pltpu.sync_copy(dst_hbm.at[pl.ds(base, RPW)], idx_v)
    pltpu.async_copy(mem_hbm.at[idx_v], rows_v, sem).wait()
    pltpu.sync_copy(rows_v, dstmem_out.at[pl.ds(base, RPW)])


def _dense_fwd_body(src_ref, dst_ref, ef_ref, dt_ref,
                    w1s_ref, w1d_ref, w1e_ref, w1t_ref, b1_ref,
                    w2_ref, b2_ref,
                    wih_ref, bih_ref, whh_ref, bhh_ref,
                    e1d_ref, e1e_ref, eb1_ref, e2_ref, eb2_ref,
                    c1_ref, cb1_ref, c2_ref, cb2_ref,
                    out_ref, newsrc_ref, gi_ref):
    src = src_ref[...]
    dst = dst_ref[...]
    ef = ef_ref[...]
    dt = dt_ref[...]

    h1 = (jnp.dot(src, w1s_ref[...], preferred_element_type=jnp.float32)
          + jnp.dot(dst, w1d_ref[...], preferred_element_type=jnp.float32)
          + jnp.dot(ef, w1e_ref[...], preferred_element_type=jnp.float32)
          + dt * w1t_ref[...]
          + b1_ref[...])
    h1 = jnp.maximum(h1, 0.0)
    msg = jnp.dot(h1, w2_ref[...], preferred_element_type=jnp.float32) + b2_ref[...]

    gi = jnp.dot(msg, wih_ref[...], preferred_element_type=jnp.float32) + bih_ref[...]
    gi_ref[...] = gi

    gh = jnp.dot(src, whh_ref[...], preferred_element_type=jnp.float32) + bhh_ref[...]
    i_r, i_z, i_n = gi[:, :MD], gi[:, MD:2 * MD], gi[:, 2 * MD:]
    h_r, h_z, h_n = gh[:, :MD], gh[:, MD:2 * MD], gh[:, 2 * MD:]
    r = jax.nn.sigmoid(i_r + h_r)
    z = jax.nn.sigmoid(i_z + h_z)
    n = jnp.tanh(i_n + r * h_n)
    newsrc_ref[...] = (1.0 - z) * n + z * src

    eh = (jnp.dot(dst, e1d_ref[...], preferred_element_type=jnp.float32)
          + jnp.dot(ef, e1e_ref[...], preferred_element_type=jnp.float32)
          + eb1_ref[...])
    eh = jnp.maximum(eh, 0.0)
    embed = jnp.dot(eh, e2_ref[...], preferred_element_type=jnp.float32) + eb2_ref[...]

    ch = jnp.maximum(jnp.dot(embed, c1_ref[...], preferred_element_type=jnp.float32)
                     + cb1_ref[...], 0.0)
    logit = jnp.dot(ch, c2_ref[...], preferred_element_type=jnp.float32) + cb2_ref[...]
    out_ref[...] = jax.nn.sigmoid(logit)


def _gru_dst_body(gi_ref, curd_ref, whh_ref, bhh_ref, newdst_ref):
    gi = gi_ref[...]
    h = curd_ref[...]
    gh = jnp.dot(h, whh_ref[...], preferred_element_type=jnp.float32) + bhh_ref[...]
    i_r, i_z, i_n = gi[:, :MD], gi[:, MD:2 * MD], gi[:, 2 * MD:]
    h_r, h_z, h_n = gh[:, :MD], gh[:, MD:2 * MD], gh[:, 2 * MD:]
    r = jax.nn.sigmoid(i_r + h_r)
    z = jax.nn.sigmoid(i_z + h_z)
    n = jnp.tanh(i_n + r * h_n)
    newdst_ref[...] = (1.0 - z) * n + z * h


def _row_spec(d):
    return pl.BlockSpec((BLK, d), lambda i: (i, 0))


def _full_spec(shape):
    return pl.BlockSpec(shape, lambda i: tuple(0 for _ in shape))


@functools.partial(jax.jit, static_argnums=())
def _dense_fwd(src_mem, dst_mem, ef, dt, weights):
    (w1s, w1d, w1e, w1t, b1, w2, b2, wih, bih, whh, bhh,
     e1d, e1e, eb1, e2, eb2, c1, cb1, c2, cb2) = weights
    grid = (B // BLK,)
    out, newsrc, gi = pl.pallas_call(
        _dense_fwd_body,
        grid=grid,
        in_specs=[
            _row_spec(MD), _row_spec(MD), _row_spec(ED), _row_spec(1),
            _full_spec(w1s.shape), _full_spec(w1d.shape), _full_spec(w1e.shape),
            _full_spec(w1t.shape), _full_spec(b1.shape),
            _full_spec(w2.shape), _full_spec(b2.shape),
            _full_spec(wih.shape), _full_spec(bih.shape),
            _full_spec(whh.shape), _full_spec(bhh.shape),
            _full_spec(e1d.shape), _full_spec(e1e.shape), _full_spec(eb1.shape),
            _full_spec(e2.shape), _full_spec(eb2.shape),
            _full_spec(c1.shape), _full_spec(cb1.shape),
            _full_spec(c2.shape), _full_spec(cb2.shape),
        ],
        out_specs=[_row_spec(1), _row_spec(MD), _row_spec(3 * MD)],
        out_shape=[
            jax.ShapeDtypeStruct((B, 1), jnp.float32),
            jax.ShapeDtypeStruct((B, MD), jnp.float32),
            jax.ShapeDtypeStruct((B, 3 * MD), jnp.float32),
        ],
    )(src_mem, dst_mem, ef, dt,
      w1s, w1d, w1e, w1t, b1, w2, b2, wih, bih, whh, bhh,
      e1d, e1e, eb1, e2, eb2, c1, cb1, c2, cb2)
    return out, newsrc, gi


def _gru_dst(gi, cur_d, whh_t, bhh):
    grid = (B // BLK,)
    return pl.pallas_call(
        _gru_dst_body,
        grid=grid,
        in_specs=[
            _row_spec(3 * MD), _row_spec(MD),
            _full_spec(whh_t.shape), _full_spec(bhh.shape),
        ],
        out_specs=_row_spec(MD),
        out_shape=jax.ShapeDtypeStruct((B, MD), jnp.float32),
    )(gi, cur_d, whh_t, bhh)


def kernel(src_ids, dst_ids, edge_feat, delta_t, memory,
           msg_W1, msg_b1, msg_W2, msg_b2,
           gru_Wih, gru_Whh, gru_bih, gru_bhh,
           emb_W1, emb_b1, emb_W2, emb_b2,
           cls_W1, cls_b1, cls_W2, cls_b2):
    w1 = msg_W1.T
    w1s, w1d = w1[:MD], w1[MD:2 * MD]
    w1e, w1t = w1[2 * MD:2 * MD + ED], w1[2 * MD + ED:]
    weights = (w1s, w1d, w1e, w1t, msg_b1[None, :],
               msg_W2.T, msg_b2[None, :],
               gru_Wih.T, gru_bih[None, :], gru_Whh.T, gru_bhh[None, :],
               emb_W1.T[:MD], emb_W1.T[MD:], emb_b1[None, :],
               emb_W2.T, emb_b2[None, :],
               cls_W1.T, cls_b1[None, :], cls_W2.T, cls_b2[None, :])

    src_mem, dst_mem = _sc_gather2(memory, src_ids, dst_ids)

    out, new_src, gi = _dense_fwd(src_mem, dst_mem, edge_feat, delta_t, weights)

    mem2 = memory.at[src_ids].set(new_src)
    cur_d = jnp.take(mem2, dst_ids, axis=0)
    new_dst = _gru_dst(gi, cur_d, gru_Whh.T, gru_bhh[None, :])
    mem3 = mem2.at[dst_ids].set(new_dst)
    return out[:, 0], mem3
